# baseline (speedup 1.0000x reference)
import functools

import jax
import jax.numpy as jnp
from jax.experimental import pallas as pl
from jax.experimental.pallas import tpu as pltpu

S, H, E, L, Hh = 2048, 768, 8, 3, 384
TB = 256
NB = S // TB


def _dot(a, b):
    return jax.lax.dot_general(a, b, (((1,), (0,)), ((), ())),
                               preferred_element_type=jnp.float32)


def _level_body(x_ref, W1_ref, b1_ref, W2_ref, b2_ref, a_ref, hw_ref,
                T_ref, bb_ref, fin_in_ref, out_ref, fin_out_ref,
                top_scr, acc_scr, Ts_scr, *, l):
    e = pl.program_id(0)
    b = pl.program_id(1)
    rows = pl.ds(b * TB, TB)
    x = x_ref[rows, :]
    a = a_ref[e]

    @pl.when(e == 0)
    def _():
        h = jnp.maximum(_dot(x, W1_ref[...]) + b1_ref[...], 0.0)
        scores = _dot(h, W2_ref[...]) + b2_ref[...]
        probs = jax.nn.softmax(scores, axis=-1)
        pmax = jnp.max(probs, axis=1, keepdims=True)
        idx = jax.lax.broadcasted_iota(jnp.int32, (TB, E), 1)
        top_scr[rows, :] = jnp.min(jnp.where(probs == pmax, idx, E), axis=1,
                                   keepdims=True)

    @pl.when(b == 0)
    def _():
        Ts_scr[...] = a * T_ref[0]

    mask = (top_scr[rows, :] == e).astype(jnp.float32)
    contrib = _dot(x * mask, Ts_scr[...]) + mask * (a * bb_ref[0])

    @pl.when(e == 0)
    def _():
        acc_scr[rows, :] = contrib

    @pl.when(e > 0)
    def _():
        acc_scr[rows, :] += contrib

    @pl.when(e == E - 1)
    def _():
        w0 = hw_ref[0]
        w1 = hw_ref[1]
        w2 = hw_ref[2]
        m = jnp.maximum(jnp.maximum(w0, w1), w2)
        e0 = jnp.exp(w0 - m)
        e1 = jnp.exp(w1 - m)
        e2 = jnp.exp(w2 - m)
        hw_l = (e0, e1, e2)[l] / (e0 + e1 + e2)
        out_ref[...] = acc_scr[rows, :]
        fin_out_ref[...] = fin_in_ref[...] + hw_l * acc_scr[rows, :]


def _run_level(l, x, fin, W1, b1, W2, b2, assignments, hier_w, rand_T, rand_b):
    f32 = jnp.float32
    out, fin2 = pl.pallas_call(
        functools.partial(_level_body, l=l),
        grid=(E, NB),
        in_specs=[
            pl.BlockSpec((S, H), lambda e, b: (0, 0)),
            pl.BlockSpec((H, Hh), lambda e, b: (0, 0)),
            pl.BlockSpec((1, Hh), lambda e, b: (0, 0)),
            pl.BlockSpec((Hh, E), lambda e, b: (0, 0)),
            pl.BlockSpec((1, E), lambda e, b: (0, 0)),
            pl.BlockSpec(memory_space=pltpu.SMEM),
            pl.BlockSpec(memory_space=pltpu.SMEM),
            pl.BlockSpec((1, H, H), lambda e, b: (e, 0, 0)),
            pl.BlockSpec((1, 1, H), lambda e, b: (e, 0, 0)),
            pl.BlockSpec((TB, H), lambda e, b: (b, 0)),
        ],
        out_specs=[
            pl.BlockSpec((TB, H), lambda e, b: (b, 0)),
            pl.BlockSpec((TB, H), lambda e, b: (b, 0)),
        ],
        out_shape=[jax.ShapeDtypeStruct((S, H), f32),
                   jax.ShapeDtypeStruct((S, H), f32)],
        scratch_shapes=[pltpu.VMEM((S, 1), jnp.int32),
                        pltpu.VMEM((S, H), f32),
                        pltpu.VMEM((H, H), f32)],
    )(x, W1[l], b1[l].reshape(1, Hh), W2[l], b2[l].reshape(1, E),
      assignments[l], hier_w, rand_T[l], rand_b[l].reshape(E, 1, H), fin)
    return out, fin2


def kernel(hidden_states, W1, b1, W2, b2, assignments, hier_w, rand_T, rand_b):
    x = hidden_states.reshape(S, H)
    fin = jnp.zeros((S, H), jnp.float32)
    for l in range(L):
        x, fin = _run_level(l, x, fin, W1, b1, W2, b2, assignments, hier_w,
                            rand_T, rand_b)
    return fin.reshape(1, S, H)


# speedup vs baseline: 1.6375x; 1.6375x over previous
import functools

import jax
import jax.numpy as jnp
from jax import lax
from jax.experimental import pallas as pl
from jax.experimental.pallas import tpu as pltpu
from jax.experimental.pallas import tpu_sc as plsc

S, H, E, L, Hh = 2048, 768, 8, 3, 384
TB = 256
NB = S // TB
NSTEP = 16
NW = 32
BPW = S // NW


def _dot(a, b):
    return jax.lax.dot_general(a, b, (((1,), (0,)), ((), ())),
                               preferred_element_type=jnp.float32)


def _shift_down(c, k):
    return jnp.concatenate([jnp.zeros((k,) + c.shape[1:], c.dtype), c[:-k]], 0)


def _router_body(x_ref, W1_ref, b1_ref, W2_ref, b2_ref,
                 pos_ref, eid_ref, sb_ref, se_ref, sv_ref, sf_ref):
    x = x_ref[...]
    h = jnp.maximum(_dot(x, W1_ref[...]) + b1_ref[...], 0.0)
    scores = _dot(h, W2_ref[...]) + b2_ref[...]
    probs = jax.nn.softmax(scores, axis=-1)
    pmax = jnp.max(probs, axis=1, keepdims=True)
    eidx = jax.lax.broadcasted_iota(jnp.int32, (S, E), 1)
    top = jnp.min(jnp.where(probs == pmax, eidx, E), axis=1, keepdims=True)

    onehot = (top == eidx).astype(jnp.float32)
    c = onehot
    k = 1
    while k < S:
        c = c + _shift_down(c, k)
        k *= 2
    counts = c[S - 1:S, :]
    inc = counts
    k = 1
    while k < E:
        inc = inc + jnp.concatenate(
            [jnp.zeros((1, k), jnp.float32), inc[:, :-k]], 1)
        k *= 2
    offs = inc - counts
    ends = offs + counts

    pos = jnp.sum(onehot * (offs + c - 1.0), axis=1, keepdims=True)
    pos_ref[...] = pos.astype(jnp.int32)

    p_iota = jax.lax.broadcasted_iota(jnp.int32, (S, E), 0).astype(jnp.float32)
    eid = jnp.sum((p_iota >= ends).astype(jnp.float32), axis=1, keepdims=True)
    eid_ref[...] = eid.astype(jnp.int32)

    b_lo = jax.lax.broadcasted_iota(jnp.int32, (NB, E), 0).astype(jnp.float32) * TB
    flag = ((offs < b_lo + TB) & (ends > b_lo)
            & (counts > 0.0)).astype(jnp.float32)
    inc_e = flag
    k = 1
    while k < E:
        inc_e = inc_e + jnp.concatenate(
            [jnp.zeros((NB, k), jnp.float32), inc_e[:, :-k]], 1)
        k *= 2
    rowtot = inc_e[:, E - 1:E]
    inc_b = rowtot
    k = 1
    while k < NB:
        inc_b = inc_b + _shift_down(inc_b, k)
        k *= 2
    rank = (inc_e - flag) + (inc_b - rowtot)

    t_iota = jax.lax.broadcasted_iota(jnp.int32, (NSTEP, NB, E), 0).astype(jnp.float32)
    eq = ((rank[None] == t_iota) * flag[None])
    b3 = jax.lax.broadcasted_iota(jnp.int32, (NSTEP, NB, E), 1).astype(jnp.float32)
    e3 = jax.lax.broadcasted_iota(jnp.int32, (NSTEP, NB, E), 2).astype(jnp.float32)
    sv = jnp.sum(jnp.sum(eq, axis=2, keepdims=True), axis=1)
    sb = jnp.sum(jnp.sum(eq * b3, axis=2, keepdims=True), axis=1)
    se = jnp.sum(jnp.sum(eq * e3, axis=2, keepdims=True), axis=1)
    sb = sb + (1.0 - sv) * (NB - 1)
    se = se + (1.0 - sv) * (E - 1)
    sb_i = sb.astype(jnp.int32)
    prev = jnp.concatenate([jnp.full((1, 1), -1, jnp.int32), sb_i[:-1]], 0)
    sb_ref[...] = sb_i
    se_ref[...] = se.astype(jnp.int32)
    sv_ref[...] = sv.astype(jnp.int32)
    sf_ref[...] = (sb_i != prev).astype(jnp.int32)


def _run_router(x, W1l, b1l, W2l, b2l):
    i32 = jnp.int32
    return pl.pallas_call(
        _router_body,
        out_shape=[jax.ShapeDtypeStruct((S, 1), i32),
                   jax.ShapeDtypeStruct((S, 1), i32),
                   jax.ShapeDtypeStruct((NSTEP, 1), i32),
                   jax.ShapeDtypeStruct((NSTEP, 1), i32),
                   jax.ShapeDtypeStruct((NSTEP, 1), i32),
                   jax.ShapeDtypeStruct((NSTEP, 1), i32)],
    )(x, W1l, b1l, W2l, b2l)


def _gmm_body(sb_ref, se_ref, sv_ref, sf_ref, xs_ref, T_ref, bb_ref,
              eid_ref, a_ref, out_ref):
    t = pl.program_id(0)
    e = se_ref[t]
    a = a_ref[e]
    valid = sv_ref[t]
    mask = jnp.where(valid > 0,
                     (eid_ref[...] == e).astype(jnp.float32),
                     jnp.zeros_like(eid_ref, jnp.float32))
    Ts = a * T_ref[0]
    contrib = _dot(xs_ref[...] * mask, Ts) + mask * (a * bb_ref[0])

    @pl.when(sf_ref[t] == 1)
    def _():
        out_ref[...] = contrib

    @pl.when(sf_ref[t] == 0)
    def _():
        out_ref[...] += contrib


def _run_gmm(sb, se, sv, sf, xs, Tl, bbl, eid, al):
    grid_spec = pltpu.PrefetchScalarGridSpec(
        num_scalar_prefetch=4,
        grid=(NSTEP,),
        in_specs=[
            pl.BlockSpec((TB, H), lambda t, sb, se, sv, sf: (sb[t], 0)),
            pl.BlockSpec((1, H, H), lambda t, sb, se, sv, sf: (se[t], 0, 0)),
            pl.BlockSpec((1, 1, H), lambda t, sb, se, sv, sf: (se[t], 0, 0)),
            pl.BlockSpec((TB, 1), lambda t, sb, se, sv, sf: (sb[t], 0)),
            pl.BlockSpec(memory_space=pltpu.SMEM),
        ],
        out_specs=pl.BlockSpec((TB, H), lambda t, sb, se, sv, sf: (sb[t], 0)),
    )
    return pl.pallas_call(
        _gmm_body,
        grid_spec=grid_spec,
        out_shape=jax.ShapeDtypeStruct((S, H), jnp.float32),
    )(sb, se, sv, sf, xs, Tl, bbl.reshape(E, 1, H), eid, al)


def _sc_scatter(pos2d, x):
    mesh = plsc.VectorSubcoreMesh(core_axis_name="c", subcore_axis_name="s")

    @functools.partial(
        pl.kernel, mesh=mesh,
        out_type=jax.ShapeDtypeStruct((S, H), jnp.float32),
        scratch_types=[pltpu.VMEM((BPW,), jnp.int32),
                       pltpu.VMEM((BPW, H), jnp.float32),
                       pltpu.SemaphoreType.DMA],
    )
    def k(pos_hbm, x_hbm, out_hbm, idx_v, rows_v, sem):
        wid = lax.axis_index("s") * 2 + lax.axis_index("c")
        pltpu.sync_copy(pos_hbm.at[wid], idx_v)
        pltpu.sync_copy(x_hbm.at[pl.ds(wid * BPW, BPW)], rows_v)
        pltpu.async_copy(rows_v, out_hbm.at[idx_v], sem).wait()

    return k(pos2d, x)


def _sc_gather(pos2d, os):
    mesh = plsc.VectorSubcoreMesh(core_axis_name="c", subcore_axis_name="s")

    @functools.partial(
        pl.kernel, mesh=mesh,
        out_type=jax.ShapeDtypeStruct((S, H), jnp.float32),
        scratch_types=[pltpu.VMEM((BPW,), jnp.int32),
                       pltpu.VMEM((BPW, H), jnp.float32),
                       pltpu.SemaphoreType.DMA],
    )
    def k(pos_hbm, os_hbm, out_hbm, idx_v, rows_v, sem):
        wid = lax.axis_index("s") * 2 + lax.axis_index("c")
        pltpu.sync_copy(pos_hbm.at[wid], idx_v)
        pltpu.async_copy(os_hbm.at[idx_v], rows_v, sem).wait()
        pltpu.sync_copy(rows_v, out_hbm.at[pl.ds(wid * BPW, BPW)])

    return k(pos2d, os)


def _combine_body(hw_ref, lo0_ref, lo1_ref, lo2_ref, out_ref):
    w0 = hw_ref[0]
    w1 = hw_ref[1]
    w2 = hw_ref[2]
    m = jnp.maximum(jnp.maximum(w0, w1), w2)
    e0 = jnp.exp(w0 - m)
    e1 = jnp.exp(w1 - m)
    e2 = jnp.exp(w2 - m)
    den = e0 + e1 + e2
    out_ref[...] = ((e0 / den) * lo0_ref[...] + (e1 / den) * lo1_ref[...]
                    + (e2 / den) * lo2_ref[...])


def _run_combine(hier_w, lo0, lo1, lo2):
    return pl.pallas_call(
        _combine_body,
        grid=(NB,),
        in_specs=[pl.BlockSpec(memory_space=pltpu.SMEM),
                  pl.BlockSpec((TB, H), lambda b: (b, 0)),
                  pl.BlockSpec((TB, H), lambda b: (b, 0)),
                  pl.BlockSpec((TB, H), lambda b: (b, 0))],
        out_specs=pl.BlockSpec((TB, H), lambda b: (b, 0)),
        out_shape=jax.ShapeDtypeStruct((S, H), jnp.float32),
    )(hier_w, lo0, lo1, lo2)


def kernel(hidden_states, W1, b1, W2, b2, assignments, hier_w, rand_T, rand_b):
    x = hidden_states.reshape(S, H)
    los = []
    for l in range(L):
        pos, eid, sb, se, sv, sf = _run_router(
            x, W1[l], b1[l].reshape(1, Hh), W2[l], b2[l].reshape(1, E))
        pos2d = pos.reshape(NW, BPW)
        sb = sb.reshape(NSTEP)
        se = se.reshape(NSTEP)
        sv = sv.reshape(NSTEP)
        sf = sf.reshape(NSTEP)
        xs = _sc_scatter(pos2d, x)
        os = _run_gmm(sb, se, sv, sf, xs, rand_T[l], rand_b[l], eid,
                      assignments[l])
        x = _sc_gather(pos2d, os)
        los.append(x)
    fin = _run_combine(hier_w, los[0], los[1], los[2])
    return fin.reshape(1, S, H)
